# Initial kernel scaffold; baseline (speedup 1.0000x reference)
import jax
import jax.numpy as jnp
from jax.experimental import pallas as pl


def kernel(input_ids, weight):
    raise NotImplementedError("write your pallas kernel here")


# speedup vs baseline: 18.6026x; 18.6026x over previous
import functools

import jax
import jax.numpy as jnp
from jax import lax
from jax.experimental import pallas as pl
from jax.experimental.pallas import tpu as pltpu
from jax.experimental.pallas import tpu_sc as plsc

L = 16
CH = 128
SH = 7
NC = 2
NS = 16
NW = NC * NS


def _build(B, V, D):
    assert B % (NW * L) == 0 and D % L == 0
    T = B // NW
    NVEC = T // L
    NR = T // CH + 2
    mesh = plsc.VectorSubcoreMesh(core_axis_name="c", subcore_axis_name="s")

    @functools.partial(
        pl.kernel,
        mesh=mesh,
        out_type=jax.ShapeDtypeStruct((B, D), jnp.float32),
        compiler_params=pltpu.CompilerParams(needs_layout_passes=False, use_tc_tiling_on_sc=False),
        scratch_types=[
            pltpu.VMEM((T,), jnp.int32),
            pltpu.VMEM((NR, CH), jnp.int32),
            pltpu.VMEM((NR, CH), jnp.int32),
            pltpu.VMEM((NR, CH), jnp.int32),
            pltpu.VMEM((CH, D), jnp.float32),
            pltpu.VMEM((CH, D), jnp.float32),
            pltpu.SemaphoreType.DMA,
        ],
    )
    def emb(ids_hbm, w_hbm, out_hbm, ids_v, inrow, indst, outdst, rowbuf, zbuf, sem):
        wid = lax.axis_index("s") * NC + lax.axis_index("c")
        base = wid * T
        pltpu.sync_copy(ids_hbm.at[pl.ds(base, T)], ids_v)

        def zrow(i, c):
            for j in range(D // L):
                zbuf[i, pl.ds(j * L, L)] = jnp.zeros((L,), jnp.float32)
            return c
        lax.fori_loop(0, CH, zrow, 0)

        lanes = lax.iota(jnp.int32, L)

        def compact(i, carry):
            nin, nout = carry
            ids16 = ids_v[pl.ds(i * L, L)]
            m = (ids16 >= 0) & (ids16 < V)
            mi = m.astype(jnp.int32)
            cs = jnp.cumsum(mi)
            s = jnp.sum(mi)
            pos = base + i * L + lanes
            pin = jnp.maximum(nin + cs - 1, 0)
            plsc.store_scatter(inrow, [pin >> SH, pin & (CH - 1)], ids16, mask=m)
            plsc.store_scatter(indst, [pin >> SH, pin & (CH - 1)], pos, mask=m)
            cso = jnp.cumsum(1 - mi)
            pout = jnp.maximum(nout + cso - 1, 0)
            plsc.store_scatter(outdst, [pout >> SH, pout & (CH - 1)], pos,
                               mask=jnp.logical_not(m))
            return nin + s, nout + (L - s)

        nin, nout = lax.fori_loop(0, NVEC, compact, (0, 0))

        def last_of(arr, n):
            q = jnp.full((L,), jnp.maximum(n - 1, 0), jnp.int32)
            return plsc.load_gather(arr, [q >> SH, q & (CH - 1)])

        def pad_tail(arr, n, val):
            for k in range(CH // L):
                p = n + k * L + lanes
                plsc.store_scatter(arr, [p >> SH, p & (CH - 1)], val)

        @pl.when(nin > 0)
        def _():
            pad_tail(inrow, nin, last_of(inrow, nin))
            pad_tail(indst, nin, last_of(indst, nin))
            ncin = (nin + CH - 1) >> SH

            def gbody(g, c):
                pltpu.async_copy(w_hbm.at[inrow.at[g]], rowbuf, sem).wait()
                pltpu.async_copy(rowbuf, out_hbm.at[indst.at[g]], sem).wait()
                return c
            lax.fori_loop(0, ncin, gbody, 0)

        @pl.when(nout > 0)
        def _():
            pad_tail(outdst, nout, last_of(outdst, nout))
            ncout = (nout + CH - 1) >> SH

            def zbody(g, c):
                pltpu.async_copy(zbuf, out_hbm.at[outdst.at[g]], sem).wait()
                return c
            lax.fori_loop(0, ncout, zbody, 0)

    return emb


@jax.jit
def kernel(input_ids, weight):
    B = input_ids.size
    V, D = weight.shape
    ids_flat = input_ids.reshape(B)
    out = _build(B, V, D)(ids_flat, weight)
    return out.reshape(input_ids.shape + (D,))
